# baseline (speedup 1.0000x reference)
import functools

import jax
import jax.numpy as jnp
from jax import lax
from jax.experimental import pallas as pl
from jax.experimental.pallas import tpu as pltpu
from jax.experimental.pallas import tpu_sc as plsc

_NC = 2
_NS = 16
_NW = _NC * _NS

_BB = 128
_L = 16


def _make_kernel(batch: int, fields: int, embed: int):
    n_blk = batch // _BB
    blk_per_w = n_blk // _NW
    bpw = blk_per_w * _BB
    c_tiles = embed // 8
    tile = 8 * _BB
    mesh = plsc.VectorSubcoreMesh(core_axis_name="c", subcore_axis_name="s")

    @functools.partial(
        pl.kernel,
        out_type=jax.ShapeDtypeStruct((fields * c_tiles * n_blk, tile),
                                      jnp.float32),
        mesh=mesh,
        compiler_params=pltpu.CompilerParams(use_tc_tiling_on_sc=False,
                                             needs_layout_passes=False),
        scratch_types=[
            pltpu.VMEM((fields, bpw), jnp.int32),
            pltpu.VMEM((bpw, embed), jnp.float32),
            pltpu.VMEM((bpw, embed), jnp.float32),
            pltpu.VMEM((c_tiles * tile,), jnp.float32),
            pltpu.VMEM((c_tiles * tile,), jnp.float32),
            pltpu.SemaphoreType.DMA,
            pltpu.SemaphoreType.DMA,
            pltpu.SemaphoreType.DMA,
            pltpu.SemaphoreType.DMA,
        ],
    )
    def body(tbl_hbm, xt2_hbm, out_hbm, idx_v, dst0, dst1, st0, st1,
             gs0, gs1, os0, os1):
        wid = lax.axis_index("s") * _NC + lax.axis_index("c")
        bcol = wid * bpw

        pltpu.sync_copy(xt2_hbm.at[pl.ds(0, fields), pl.ds(bcol, bpw)], idx_v)

        def fire(f, dst, sem):
            pltpu.async_copy(tbl_hbm.at[idx_v.at[f]], dst, sem)

        def drain(dst, sem):
            pltpu.make_async_copy(tbl_hbm.at[idx_v.at[0]], dst, sem).wait()

        row_vecs = [lax.iota(jnp.int32, _L) + k * _L for k in range(_BB // _L)]

        def transpose(dst, jl, stage):
            @plsc.parallel_loop(0, embed, 2, unroll=4)
            def _(c):
                for cc in range(2):
                    base = ((c + cc) // 8) * tile + lax.rem(c + cc, 8) * _BB
                    cols = jnp.full((_L,), c + cc, jnp.int32)
                    for k in range(_BB // _L):
                        vals = plsc.load_gather(
                            dst, [row_vecs[k] + jl * _BB, cols])
                        stage[pl.ds(base + k * _L, _L)] = vals

        def fire_out(f, jl, stage, sem):
            jb = wid * blk_per_w + jl
            for i in range(c_tiles):
                pltpu.async_copy(stage.at[pl.ds(i * tile, tile)],
                                 out_hbm.at[(f * c_tiles + i) * n_blk + jb],
                                 sem)

        def drain_out(stage, sem):
            for i in range(c_tiles):
                pltpu.make_async_copy(stage.at[pl.ds(i * tile, tile)],
                                      out_hbm.at[0], sem).wait()

        def blocks(f, dst, first):
            for jl in range(blk_per_w):
                stage, sem = (st0, os0) if jl % 2 == 0 else (st1, os1)
                if jl >= 2:
                    drain_out(stage, sem)
                else:
                    @pl.when(jnp.logical_not(first))
                    def _():
                        drain_out(stage, sem)

                transpose(dst, jl, stage)
                fire_out(f, jl, stage, sem)

        fire(0, dst0, gs0)

        def step(t, carry):
            fa = 2 * t
            fire(fa + 1, dst1, gs1)
            drain(dst0, gs0)
            blocks(fa, dst0, t == 0)

            @pl.when(t < fields // 2 - 1)
            def _():
                fire(fa + 2, dst0, gs0)

            drain(dst1, gs1)
            blocks(fa + 1, dst1, jnp.bool_(False))
            return carry

        lax.fori_loop(0, fields // 2, step, 0)
        drain_out(st0, os0)
        drain_out(st1, os1)

    return body


def kernel(x, table):
    batch, fields = x.shape
    vocab, embed = table.shape
    tbl128 = jnp.pad(table, ((0, 0), (0, embed)))
    tbl2 = tbl128.reshape(2 * vocab, embed)
    xt2 = (x.T * 2).astype(jnp.int32)
    n_blk = batch // _BB
    out5 = _make_kernel(batch, fields, embed)(tbl2, xt2)
    out = (out5.reshape(fields, embed // 8, n_blk, 8, _BB)
           .transpose(2, 4, 0, 1, 3)
           .reshape(batch, fields, embed))
    return out


# speedup vs baseline: 1.4241x; 1.4241x over previous
import functools

import jax
import jax.numpy as jnp
from jax import lax
from jax.experimental import pallas as pl
from jax.experimental.pallas import tpu as pltpu
from jax.experimental.pallas import tpu_sc as plsc

_NC = 2
_NS = 16
_NW = _NC * _NS

_BB = 128
_L = 16


def _make_kernel(batch: int, fields: int, embed: int):
    n_blk = batch // _BB
    blk_per_w = n_blk // _NW
    bpw = blk_per_w * _BB
    c_tiles = embed // 8
    tile = 8 * _BB
    mesh = plsc.VectorSubcoreMesh(core_axis_name="c", subcore_axis_name="s")

    @functools.partial(
        pl.kernel,
        out_type=jax.ShapeDtypeStruct((fields * c_tiles * n_blk, 8, _BB),
                                      jnp.float32),
        mesh=mesh,
        compiler_params=pltpu.CompilerParams(use_tc_tiling_on_sc=False,
                                             needs_layout_passes=False),
        scratch_types=[
            pltpu.VMEM((fields, bpw), jnp.int32),
            pltpu.VMEM((bpw, embed), jnp.float32),
            pltpu.VMEM((bpw, embed), jnp.float32),
            pltpu.VMEM((embed, _BB + 1), jnp.float32),
            pltpu.VMEM((embed, _BB + 1), jnp.float32),
            pltpu.SemaphoreType.DMA,
            pltpu.SemaphoreType.DMA,
            pltpu.SemaphoreType.DMA,
            pltpu.SemaphoreType.DMA,
        ],
    )
    def body(tbl_hbm, xt2_hbm, out_hbm, idx_v, dst0, dst1, st0, st1,
             gs0, gs1, os0, os1):
        wid = lax.axis_index("s") * _NC + lax.axis_index("c")
        bcol = wid * bpw

        pltpu.sync_copy(xt2_hbm.at[pl.ds(0, fields), pl.ds(bcol, bpw)], idx_v)

        def fire(f, dst, sem):
            pltpu.async_copy(tbl_hbm.at[idx_v.at[f]], dst, sem)

        def drain(dst, sem):
            pltpu.make_async_copy(tbl_hbm.at[idx_v.at[0]], dst, sem).wait()

        col_vecs = [lax.iota(jnp.int32, _L) + k * _L
                    for k in range(embed // _L)]

        def transpose(dst, jl, stage):
            @plsc.parallel_loop(0, _BB, 2, unroll=4)
            def _(b):
                for bb in range(2):
                    rows = jnp.full((_L,), jl * _BB + b + bb, jnp.int32)
                    outb = jnp.full((_L,), b + bb, jnp.int32)
                    for k in range(embed // _L):
                        vals = plsc.load_gather(dst, [rows, col_vecs[k]])
                        plsc.store_scatter(stage, [col_vecs[k], outb], vals)

        def fire_out(f, jl, stage, sem):
            jb = wid * blk_per_w + jl
            for i in range(c_tiles):
                pltpu.async_copy(stage.at[pl.ds(i * 8, 8), pl.ds(0, _BB)],
                                 out_hbm.at[(f * c_tiles + i) * n_blk + jb],
                                 sem)

        def drain_out(stage, sem):
            for i in range(c_tiles):
                pltpu.make_async_copy(stage.at[pl.ds(i * 8, 8), pl.ds(0, _BB)],
                                      out_hbm.at[0], sem).wait()

        def blocks(f, dst, first):
            for jl in range(blk_per_w):
                stage, sem = (st0, os0) if jl % 2 == 0 else (st1, os1)
                if jl >= 2:
                    drain_out(stage, sem)
                else:
                    @pl.when(jnp.logical_not(first))
                    def _():
                        drain_out(stage, sem)

                transpose(dst, jl, stage)
                fire_out(f, jl, stage, sem)

        fire(0, dst0, gs0)

        def step(t, carry):
            fa = 2 * t
            fire(fa + 1, dst1, gs1)
            drain(dst0, gs0)
            blocks(fa, dst0, t == 0)

            @pl.when(t < fields // 2 - 1)
            def _():
                fire(fa + 2, dst0, gs0)

            drain(dst1, gs1)
            blocks(fa + 1, dst1, jnp.bool_(False))
            return carry

        lax.fori_loop(0, fields // 2, step, 0)
        drain_out(st0, os0)
        drain_out(st1, os1)

    return body


def kernel(x, table):
    batch, fields = x.shape
    vocab, embed = table.shape
    tbl128 = jnp.pad(table, ((0, 0), (0, embed)))
    tbl2 = tbl128.reshape(2 * vocab, embed)
    xt2 = (x.T * 2).astype(jnp.int32)
    n_blk = batch // _BB
    out5 = _make_kernel(batch, fields, embed)(tbl2, xt2)
    out = (out5.reshape(fields, embed // 8, n_blk, 8, _BB)
           .transpose(2, 4, 0, 1, 3)
           .reshape(batch, fields, embed))
    return out
